# baseline (speedup 1.0000x reference)
import functools

import numpy as np
import jax
import jax.numpy as jnp
from jax import lax
from jax.experimental import pallas as pl
from jax.experimental.pallas import tpu as pltpu
from jax.experimental.pallas import tpu_sc as plsc

P_BINS = 5
T_BINS = 16
R_IN = 4
R_OUT = 4
C_IN = 4
C_OUT = 4
B = P_BINS * T_BINS
N = 10000
F_IN = C_IN * R_IN
F_OUT = C_OUT * R_OUT
K = F_IN * B
NNZ = 3200000

NC = 2
NS = 16
L = 16
NW = NC * NS
GRP = 128
CHUNK = 1024
NG = CHUNK // GRP
CPW = 98
CPW_LAST = 87
EPW = CPW * CHUNK
NPS = 624
NTAIL = N - NPS * NS
assert 31 * EPW + CPW_LAST * CHUNK == NNZ


def _rotation_table():
    p, t = P_BINS, T_BINS
    bb = p * t

    def angle_rotation(v):
        pif = v.reshape(-1, p).copy()
        pif1 = np.zeros_like(pif)
        for i in range(R_IN):
            pif1[i * t:i * t + t - 1, :] = pif[i * t + 1:i * t + t, :]
            pif1[i * t + t - 1, :] = pif[i * t, :]
        return pif1.reshape(-1)

    def kernel_rotation(v):
        pif = v.reshape(-1, t * p)
        pif1 = np.zeros_like(pif)
        pif1[1:, :] = pif[:-1, :]
        pif1[0, :] = pif[-1, :]
        return pif1.reshape(-1)

    small = np.zeros((R_IN * bb, R_OUT), dtype=np.int64)
    pif = np.arange(R_IN * bb, dtype=np.int64)
    small[:, 0] = pif
    for j in range(1, R_OUT):
        small[:, j] = kernel_rotation(angle_rotation(pif))
        pif = small[:, j]
    rot = np.zeros((C_IN * R_IN * bb, R_OUT), dtype=np.int64)
    for i in range(C_IN):
        rot[i * R_IN * bb:(i + 1) * R_IN * bb] = small + i * R_IN * bb
    return rot


def _m_index_table():
    rot = _rotation_table()
    idx = np.zeros((F_IN, K), dtype=np.int32)
    for f in range(F_IN):
        for b in range(B):
            for i in range(C_OUT):
                for j in range(R_OUT):
                    idx[f, b * F_OUT + i * R_OUT + j] = i * K + rot[b * F_IN + f, j]
    return idx


def _build_m(weights):
    us = []
    u = weights
    for _ in range(R_OUT):
        us.append(u)
        v = u.reshape(C_OUT, C_IN, R_IN, T_BINS, P_BINS)
        v = jnp.roll(v, -1, axis=3)
        v = jnp.roll(v, 1, axis=2)
        u = v.reshape(C_OUT, K)
    ubf = jnp.stack(us).reshape(R_OUT, C_OUT, B, F_IN)
    return jnp.transpose(ubf, (3, 2, 1, 0)).reshape(F_IN, K)


def _mm_body(x_ref, m_ref, o_ref):
    o_ref[...] = jnp.dot(x_ref[...], m_ref[...],
                         preferred_element_type=jnp.float32)


def _dense_stage(x, m):
    return pl.pallas_call(
        _mm_body,
        grid=(10,),
        in_specs=[
            pl.BlockSpec((1000, F_IN), lambda i: (i, 0)),
            pl.BlockSpec((F_IN, K), lambda i: (0, 0)),
        ],
        out_specs=pl.BlockSpec((1000, K), lambda i: (i, 0)),
        out_shape=jax.ShapeDtypeStruct((N, K), jnp.float32),
    )(x, m)


def _add_body(p_ref, o_ref):
    o_ref[...] = p_ref[0] + p_ref[1]


def _combine_stage(partials):
    return pl.pallas_call(
        _add_body,
        out_shape=jax.ShapeDtypeStruct((N, F_OUT), jnp.float32),
    )(partials)


def _sc_body(g2_hbm, rows_hbm, cols_hbm, vals_hbm, zeros_hbm, out_hbm,
             r_v, c_v, v_v, g_v, n_v, rows_v, acc_sh, rcv_sem, g_sem, s_sem):
    cid = lax.axis_index("c")
    sid = lax.axis_index("s")
    wid = sid * NC + cid

    pltpu.sync_copy(zeros_hbm.at[pl.ds(sid * NPS, NPS)],
                    acc_sh.at[pl.ds(sid * NPS, NPS)])

    @pl.when(sid == 0)
    def _():
        pltpu.sync_copy(zeros_hbm.at[pl.ds(NPS * NS, NTAIL)],
                        acc_sh.at[pl.ds(NPS * NS, NTAIL)])

    plsc.subcore_barrier()

    base = wid * EPW
    nch = jnp.where(wid == NW - 1, CPW_LAST, CPW)

    def issue_rcv(t, p):
        off = base + t * CHUNK
        pltpu.async_copy(rows_hbm.at[pl.ds(off, CHUNK)], r_v.at[p],
                         rcv_sem.at[p])
        pltpu.async_copy(cols_hbm.at[pl.ds(off, CHUNK)], c_v.at[p],
                         rcv_sem.at[p])
        pltpu.async_copy(vals_hbm.at[pl.ds(off, CHUNK)], v_v.at[p],
                         rcv_sem.at[p])

    def wait_rcv(p):
        pltpu.make_async_copy(rows_hbm.at[pl.ds(0, CHUNK)], r_v.at[p],
                              rcv_sem.at[p]).wait()
        pltpu.make_async_copy(cols_hbm.at[pl.ds(0, CHUNK)], c_v.at[p],
                              rcv_sem.at[p]).wait()
        pltpu.make_async_copy(vals_hbm.at[pl.ds(0, CHUNK)], v_v.at[p],
                              rcv_sem.at[p]).wait()

    def idx_compute(p, i):
        r = r_v[p, pl.ds(i * L, L)]
        c = c_v[p, pl.ds(i * L, L)]
        q1 = lax.shift_right_logical(r, 4)
        n = (q1.astype(jnp.float32) * 0.2).astype(jnp.int32)
        g = c * B + (r - n * B)
        g_v[p, i // NG, pl.ds((i % NG) * L, L)] = g
        n_v[p, i // NG, pl.ds((i % NG) * L, L)] = n

    def scale_compute(p, i):
        vv = v_v[p, pl.ds(i * L, L)]
        for e in range(L):
            edge = i * L + e
            eb = jnp.broadcast_to(i * 0 + e, (L,)).astype(jnp.int32)
            sp = jnp.take_along_axis(vv, eb, axis=0)
            rows_v[p, edge, :] = rows_v[p, edge, :] * sp

    def idx_pass(p):
        @plsc.parallel_loop(0, CHUNK // L, unroll=2)
        def _(i):
            idx_compute(p, i)

    def fused_pass(pscale, pidx):
        @plsc.parallel_loop(0, CHUNK // L, unroll=2)
        def _(i):
            idx_compute(pidx, i)
            scale_compute(pscale, i)

    def fire_gathers(p):
        for j in range(NG):
            pltpu.async_copy(g2_hbm.at[g_v.at[p, j]],
                             rows_v.at[p, pl.ds(j * GRP, GRP)],
                             g_sem.at[p])

    def drain_gathers(p):
        pltpu.make_async_copy(g2_hbm.at[pl.ds(0, CHUNK)],
                              rows_v.at[p],
                              g_sem.at[p]).wait()

    def scale_pass(p):
        @plsc.parallel_loop(0, CHUNK // L, unroll=3)
        def _(i):
            vv = v_v[p, pl.ds(i * L, L)]
            for e in range(L):
                edge = i * L + e
                eb = jnp.broadcast_to(i * 0 + e, (L,)).astype(jnp.int32)
                sp = jnp.take_along_axis(vv, eb, axis=0)
                rows_v[p, edge, :] = rows_v[p, edge, :] * sp

    def fire_scatter(p):
        for j in range(NG):
            pltpu.async_copy(rows_v.at[p, pl.ds(j * GRP, GRP)],
                             acc_sh.at[n_v.at[p, j]],
                             s_sem.at[p], add=True)

    def drain_scatter(p):
        pltpu.make_async_copy(g2_hbm.at[pl.ds(0, CHUNK)],
                              rows_v.at[p],
                              s_sem.at[p]).wait()

    issue_rcv(0, 0)
    wait_rcv(0)
    idx_pass(0)
    fire_gathers(0)
    issue_rcv(1, 1)

    def loop_body(t, _):
        p = lax.rem(t, 2)
        q = 1 - p

        @pl.when(t >= 1)
        def _():
            drain_scatter(q)

        @pl.when(t < nch - 1)
        def _():
            wait_rcv(q)
            idx_pass(q)

        drain_gathers(p)

        @pl.when(t < nch - 1)
        def _():
            fire_gathers(q)

        scale_pass(p)
        fire_scatter(p)

        @pl.when(t < nch - 2)
        def _():
            issue_rcv(t + 2, p)
        return 0

    lax.fori_loop(0, nch, loop_body, 0)
    drain_scatter(lax.rem(nch - 1, 2))

    plsc.subcore_barrier()
    pltpu.sync_copy(acc_sh.at[pl.ds(sid * NPS, NPS)],
                    out_hbm.at[cid, pl.ds(sid * NPS, NPS)])

    @pl.when(sid == 0)
    def _():
        pltpu.sync_copy(acc_sh.at[pl.ds(NPS * NS, NTAIL)],
                        out_hbm.at[cid, pl.ds(NPS * NS, NTAIL)])


@functools.partial(
    pl.kernel,
    out_type=jax.ShapeDtypeStruct((NC, N, F_OUT), jnp.float32),
    mesh=plsc.VectorSubcoreMesh(core_axis_name="c", subcore_axis_name="s"),
    compiler_params=pltpu.CompilerParams(needs_layout_passes=False,
                                         use_tc_tiling_on_sc=False),
    scratch_types=[
        pltpu.VMEM((2, CHUNK), jnp.int32),
        pltpu.VMEM((2, CHUNK), jnp.int32),
        pltpu.VMEM((2, CHUNK), jnp.float32),
        pltpu.VMEM((2, NG, GRP), jnp.int32),
        pltpu.VMEM((2, NG, GRP), jnp.int32),
        pltpu.VMEM((2, CHUNK, F_OUT), jnp.float32),
        pltpu.VMEM_SHARED((N, F_OUT), jnp.float32),
        pltpu.SemaphoreType.DMA((2,)),
        pltpu.SemaphoreType.DMA((2,)),
        pltpu.SemaphoreType.DMA((2,)),
    ],
)
def _sparse_stage(g2_hbm, rows_hbm, cols_hbm, vals_hbm, zeros_hbm, out_hbm,
                  r_v, c_v, v_v, g_v, n_v, rows_v, acc_sh,
                  rcv_sem, g_sem, s_sem):
    _sc_body(g2_hbm, rows_hbm, cols_hbm, vals_hbm, zeros_hbm, out_hbm,
             r_v, c_v, v_v, g_v, n_v, rows_v, acc_sh, rcv_sem, g_sem, s_sem)


def kernel(x, conn_rows, conn_cols, conn_values, weights):
    m = _build_m(weights)
    g2 = _dense_stage(x, m).reshape(N * B, F_OUT)

    zeros = jnp.zeros((N, F_OUT), jnp.float32)
    partials = _sparse_stage(g2,
                             conn_rows.astype(jnp.int32),
                             conn_cols.astype(jnp.int32),
                             conn_values.astype(jnp.float32),
                             zeros)
    return _combine_stage(partials)


# speedup vs baseline: 1.0054x; 1.0054x over previous
import functools

import numpy as np
import jax
import jax.numpy as jnp
from jax import lax
from jax.experimental import pallas as pl
from jax.experimental.pallas import tpu as pltpu
from jax.experimental.pallas import tpu_sc as plsc

P_BINS = 5
T_BINS = 16
R_IN = 4
R_OUT = 4
C_IN = 4
C_OUT = 4
B = P_BINS * T_BINS
N = 10000
F_IN = C_IN * R_IN
F_OUT = C_OUT * R_OUT
K = F_IN * B
NNZ = 3200000

NC = 2
NS = 16
L = 16
NW = NC * NS
GRP = 128
CHUNK = 1024
NG = CHUNK // GRP
CPW = 98
CPW_LAST = 87
EPW = CPW * CHUNK
NPS = 624
NTAIL = N - NPS * NS
assert 31 * EPW + CPW_LAST * CHUNK == NNZ


def _rotation_table():
    p, t = P_BINS, T_BINS
    bb = p * t

    def angle_rotation(v):
        pif = v.reshape(-1, p).copy()
        pif1 = np.zeros_like(pif)
        for i in range(R_IN):
            pif1[i * t:i * t + t - 1, :] = pif[i * t + 1:i * t + t, :]
            pif1[i * t + t - 1, :] = pif[i * t, :]
        return pif1.reshape(-1)

    def kernel_rotation(v):
        pif = v.reshape(-1, t * p)
        pif1 = np.zeros_like(pif)
        pif1[1:, :] = pif[:-1, :]
        pif1[0, :] = pif[-1, :]
        return pif1.reshape(-1)

    small = np.zeros((R_IN * bb, R_OUT), dtype=np.int64)
    pif = np.arange(R_IN * bb, dtype=np.int64)
    small[:, 0] = pif
    for j in range(1, R_OUT):
        small[:, j] = kernel_rotation(angle_rotation(pif))
        pif = small[:, j]
    rot = np.zeros((C_IN * R_IN * bb, R_OUT), dtype=np.int64)
    for i in range(C_IN):
        rot[i * R_IN * bb:(i + 1) * R_IN * bb] = small + i * R_IN * bb
    return rot


def _m_index_table():
    rot = _rotation_table()
    idx = np.zeros((F_IN, K), dtype=np.int32)
    for f in range(F_IN):
        for b in range(B):
            for i in range(C_OUT):
                for j in range(R_OUT):
                    idx[f, b * F_OUT + i * R_OUT + j] = i * K + rot[b * F_IN + f, j]
    return idx


def _build_m(weights):
    us = []
    u = weights
    for _ in range(R_OUT):
        us.append(u)
        v = u.reshape(C_OUT, C_IN, R_IN, T_BINS, P_BINS)
        v = jnp.roll(v, -1, axis=3)
        v = jnp.roll(v, 1, axis=2)
        u = v.reshape(C_OUT, K)
    ubf = jnp.stack(us).reshape(R_OUT, C_OUT, B, F_IN)
    return jnp.transpose(ubf, (3, 2, 1, 0)).reshape(F_IN, K)


def _mm_body(x_ref, m_ref, o_ref):
    o_ref[...] = jnp.dot(x_ref[...], m_ref[...],
                         preferred_element_type=jnp.float32)


def _dense_stage(x, m):
    return pl.pallas_call(
        _mm_body,
        grid=(10,),
        in_specs=[
            pl.BlockSpec((1000, F_IN), lambda i: (i, 0)),
            pl.BlockSpec((F_IN, K), lambda i: (0, 0)),
        ],
        out_specs=pl.BlockSpec((1000, K), lambda i: (i, 0)),
        out_shape=jax.ShapeDtypeStruct((N, K), jnp.float32),
    )(x, m)


def _add_body(p_ref, o_ref):
    o_ref[...] = p_ref[0] + p_ref[1]


def _combine_stage(partials):
    return pl.pallas_call(
        _add_body,
        out_shape=jax.ShapeDtypeStruct((N, F_OUT), jnp.float32),
    )(partials)


def _sc_body(g2_hbm, rows_hbm, cols_hbm, vals_hbm, zeros_hbm, out_hbm,
             r_v, c_v, v_v, g_v, n_v, rows_v, acc_sh, rcv_sem, g_sem, s_sem):
    cid = lax.axis_index("c")
    sid = lax.axis_index("s")
    wid = sid * NC + cid

    pltpu.sync_copy(zeros_hbm.at[pl.ds(sid * NPS, NPS)],
                    acc_sh.at[pl.ds(sid * NPS, NPS)])

    @pl.when(sid == 0)
    def _():
        pltpu.sync_copy(zeros_hbm.at[pl.ds(NPS * NS, NTAIL)],
                        acc_sh.at[pl.ds(NPS * NS, NTAIL)])

    plsc.subcore_barrier()

    base = wid * EPW
    nch = jnp.where(wid == NW - 1, CPW_LAST, CPW)

    def issue_rcv(t, p):
        off = base + t * CHUNK
        pltpu.async_copy(rows_hbm.at[pl.ds(off, CHUNK)], r_v.at[p],
                         rcv_sem.at[p])
        pltpu.async_copy(cols_hbm.at[pl.ds(off, CHUNK)], c_v.at[p],
                         rcv_sem.at[p])
        pltpu.async_copy(vals_hbm.at[pl.ds(off, CHUNK)], v_v.at[p],
                         rcv_sem.at[p])

    def wait_rcv(p):
        pltpu.make_async_copy(rows_hbm.at[pl.ds(0, CHUNK)], r_v.at[p],
                              rcv_sem.at[p]).wait()
        pltpu.make_async_copy(cols_hbm.at[pl.ds(0, CHUNK)], c_v.at[p],
                              rcv_sem.at[p]).wait()
        pltpu.make_async_copy(vals_hbm.at[pl.ds(0, CHUNK)], v_v.at[p],
                              rcv_sem.at[p]).wait()

    def idx_compute(p, i):
        r = r_v[p, pl.ds(i * L, L)]
        c = c_v[p, pl.ds(i * L, L)]
        q1 = lax.shift_right_logical(r, 4)
        n = (q1.astype(jnp.float32) * 0.2).astype(jnp.int32)
        g = c * B + (r - n * B)
        g_v[p, i // NG, pl.ds((i % NG) * L, L)] = g
        n_v[p, i // NG, pl.ds((i % NG) * L, L)] = n

    def idx_pass(p):
        @plsc.parallel_loop(0, CHUNK // L, unroll=2)
        def _(i):
            idx_compute(p, i)

    def fire_gathers(p):
        for j in range(NG):
            pltpu.async_copy(g2_hbm.at[g_v.at[p, j]],
                             rows_v.at[p, pl.ds(j * GRP, GRP)],
                             g_sem.at[p])

    def drain_gathers(p):
        pltpu.make_async_copy(g2_hbm.at[pl.ds(0, CHUNK)],
                              rows_v.at[p],
                              g_sem.at[p]).wait()

    def scale_pass(p):
        @plsc.parallel_loop(0, CHUNK // L, unroll=2)
        def _(i):
            vv = v_v[p, pl.ds(i * L, L)]
            for e in range(L):
                edge = i * L + e
                eb = jnp.broadcast_to(i * 0 + e, (L,)).astype(jnp.int32)
                sp = jnp.take_along_axis(vv, eb, axis=0)
                rows_v[p, edge, :] = rows_v[p, edge, :] * sp

    def fire_scatter(p):
        for j in range(NG):
            pltpu.async_copy(rows_v.at[p, pl.ds(j * GRP, GRP)],
                             acc_sh.at[n_v.at[p, j]],
                             s_sem.at[p], add=True)

    def drain_scatter(p):
        pltpu.make_async_copy(g2_hbm.at[pl.ds(0, CHUNK)],
                              rows_v.at[p],
                              s_sem.at[p]).wait()

    issue_rcv(0, 0)
    wait_rcv(0)
    idx_pass(0)
    fire_gathers(0)
    issue_rcv(1, 1)

    def loop_body(t, _):
        p = lax.rem(t, 2)
        q = 1 - p

        @pl.when(t >= 1)
        def _():
            drain_scatter(q)

        @pl.when(t < nch - 1)
        def _():
            wait_rcv(q)
            idx_pass(q)

        drain_gathers(p)

        @pl.when(t < nch - 1)
        def _():
            fire_gathers(q)

        scale_pass(p)
        fire_scatter(p)

        @pl.when(t < nch - 2)
        def _():
            issue_rcv(t + 2, p)
        return 0

    lax.fori_loop(0, nch, loop_body, 0)
    drain_scatter(lax.rem(nch - 1, 2))

    plsc.subcore_barrier()
    pltpu.sync_copy(acc_sh.at[pl.ds(sid * NPS, NPS)],
                    out_hbm.at[cid, pl.ds(sid * NPS, NPS)])

    @pl.when(sid == 0)
    def _():
        pltpu.sync_copy(acc_sh.at[pl.ds(NPS * NS, NTAIL)],
                        out_hbm.at[cid, pl.ds(NPS * NS, NTAIL)])


@functools.partial(
    pl.kernel,
    out_type=jax.ShapeDtypeStruct((NC, N, F_OUT), jnp.float32),
    mesh=plsc.VectorSubcoreMesh(core_axis_name="c", subcore_axis_name="s"),
    compiler_params=pltpu.CompilerParams(needs_layout_passes=False,
                                         use_tc_tiling_on_sc=False),
    scratch_types=[
        pltpu.VMEM((2, CHUNK), jnp.int32),
        pltpu.VMEM((2, CHUNK), jnp.int32),
        pltpu.VMEM((2, CHUNK), jnp.float32),
        pltpu.VMEM((2, NG, GRP), jnp.int32),
        pltpu.VMEM((2, NG, GRP), jnp.int32),
        pltpu.VMEM((2, CHUNK, F_OUT), jnp.float32),
        pltpu.VMEM_SHARED((N, F_OUT), jnp.float32),
        pltpu.SemaphoreType.DMA((2,)),
        pltpu.SemaphoreType.DMA((2,)),
        pltpu.SemaphoreType.DMA((2,)),
    ],
)
def _sparse_stage(g2_hbm, rows_hbm, cols_hbm, vals_hbm, zeros_hbm, out_hbm,
                  r_v, c_v, v_v, g_v, n_v, rows_v, acc_sh,
                  rcv_sem, g_sem, s_sem):
    _sc_body(g2_hbm, rows_hbm, cols_hbm, vals_hbm, zeros_hbm, out_hbm,
             r_v, c_v, v_v, g_v, n_v, rows_v, acc_sh, rcv_sem, g_sem, s_sem)


def kernel(x, conn_rows, conn_cols, conn_values, weights):
    m = _build_m(weights)
    g2 = _dense_stage(x, m).reshape(N * B, F_OUT)

    zeros = jnp.zeros((N, F_OUT), jnp.float32)
    partials = _sparse_stage(g2,
                             conn_rows.astype(jnp.int32),
                             conn_cols.astype(jnp.int32),
                             conn_values.astype(jnp.float32),
                             zeros)
    return _combine_stage(partials)


# speedup vs baseline: 1.0066x; 1.0011x over previous
import functools

import numpy as np
import jax
import jax.numpy as jnp
from jax import lax
from jax.experimental import pallas as pl
from jax.experimental.pallas import tpu as pltpu
from jax.experimental.pallas import tpu_sc as plsc

P_BINS = 5
T_BINS = 16
R_IN = 4
R_OUT = 4
C_IN = 4
C_OUT = 4
B = P_BINS * T_BINS
N = 10000
F_IN = C_IN * R_IN
F_OUT = C_OUT * R_OUT
K = F_IN * B
NNZ = 3200000

NC = 2
NS = 16
L = 16
NW = NC * NS
GRP = 128
CHUNK = 1024
NG = CHUNK // GRP
CPW = 98
CPW_LAST = 87
EPW = CPW * CHUNK
NPS = 624
NTAIL = N - NPS * NS
assert 31 * EPW + CPW_LAST * CHUNK == NNZ


def _rotation_table():
    p, t = P_BINS, T_BINS
    bb = p * t

    def angle_rotation(v):
        pif = v.reshape(-1, p).copy()
        pif1 = np.zeros_like(pif)
        for i in range(R_IN):
            pif1[i * t:i * t + t - 1, :] = pif[i * t + 1:i * t + t, :]
            pif1[i * t + t - 1, :] = pif[i * t, :]
        return pif1.reshape(-1)

    def kernel_rotation(v):
        pif = v.reshape(-1, t * p)
        pif1 = np.zeros_like(pif)
        pif1[1:, :] = pif[:-1, :]
        pif1[0, :] = pif[-1, :]
        return pif1.reshape(-1)

    small = np.zeros((R_IN * bb, R_OUT), dtype=np.int64)
    pif = np.arange(R_IN * bb, dtype=np.int64)
    small[:, 0] = pif
    for j in range(1, R_OUT):
        small[:, j] = kernel_rotation(angle_rotation(pif))
        pif = small[:, j]
    rot = np.zeros((C_IN * R_IN * bb, R_OUT), dtype=np.int64)
    for i in range(C_IN):
        rot[i * R_IN * bb:(i + 1) * R_IN * bb] = small + i * R_IN * bb
    return rot


def _m_index_table():
    rot = _rotation_table()
    idx = np.zeros((F_IN, K), dtype=np.int32)
    for f in range(F_IN):
        for b in range(B):
            for i in range(C_OUT):
                for j in range(R_OUT):
                    idx[f, b * F_OUT + i * R_OUT + j] = i * K + rot[b * F_IN + f, j]
    return idx


def _build_m(weights):
    us = []
    u = weights
    for _ in range(R_OUT):
        us.append(u)
        v = u.reshape(C_OUT, C_IN, R_IN, T_BINS, P_BINS)
        v = jnp.roll(v, -1, axis=3)
        v = jnp.roll(v, 1, axis=2)
        u = v.reshape(C_OUT, K)
    ubf = jnp.stack(us).reshape(R_OUT, C_OUT, B, F_IN)
    return jnp.transpose(ubf, (3, 2, 1, 0)).reshape(F_IN, K)


def _mm_body(x_ref, m_ref, o_ref):
    o_ref[...] = jnp.dot(x_ref[...], m_ref[...],
                         preferred_element_type=jnp.float32)


def _dense_stage(x, m):
    return pl.pallas_call(
        _mm_body,
        grid=(10,),
        in_specs=[
            pl.BlockSpec((1000, F_IN), lambda i: (i, 0)),
            pl.BlockSpec((F_IN, K), lambda i: (0, 0)),
        ],
        out_specs=pl.BlockSpec((1000, K), lambda i: (i, 0)),
        out_shape=jax.ShapeDtypeStruct((N, K), jnp.float32),
    )(x, m)


def _add_body(p_ref, o_ref):
    o_ref[...] = p_ref[0] + p_ref[1]


def _combine_stage(partials):
    return pl.pallas_call(
        _add_body,
        out_shape=jax.ShapeDtypeStruct((N, F_OUT), jnp.float32),
    )(partials)


def _sc_body(g2_hbm, rows_hbm, cols_hbm, vals_hbm, zeros_hbm, out_hbm,
             r_v, c_v, v_v, g_v, n_v, rows_v, acc_sh, rcv_sem, g_sem, s_sem):
    cid = lax.axis_index("c")
    sid = lax.axis_index("s")
    wid = sid * NC + cid

    pltpu.sync_copy(zeros_hbm.at[pl.ds(sid * NPS, NPS)],
                    acc_sh.at[pl.ds(sid * NPS, NPS)])

    @pl.when(sid == 0)
    def _():
        pltpu.sync_copy(zeros_hbm.at[pl.ds(NPS * NS, NTAIL)],
                        acc_sh.at[pl.ds(NPS * NS, NTAIL)])

    plsc.subcore_barrier()

    base = wid * EPW
    nch = jnp.where(wid == NW - 1, CPW_LAST, CPW)

    def issue_rcv(t, p):
        off = base + t * CHUNK
        pltpu.async_copy(rows_hbm.at[pl.ds(off, CHUNK)], r_v.at[p],
                         rcv_sem.at[p])
        pltpu.async_copy(cols_hbm.at[pl.ds(off, CHUNK)], c_v.at[p],
                         rcv_sem.at[p])
        pltpu.async_copy(vals_hbm.at[pl.ds(off, CHUNK)], v_v.at[p],
                         rcv_sem.at[p])

    def wait_rcv(p):
        pltpu.make_async_copy(rows_hbm.at[pl.ds(0, CHUNK)], r_v.at[p],
                              rcv_sem.at[p]).wait()
        pltpu.make_async_copy(cols_hbm.at[pl.ds(0, CHUNK)], c_v.at[p],
                              rcv_sem.at[p]).wait()
        pltpu.make_async_copy(vals_hbm.at[pl.ds(0, CHUNK)], v_v.at[p],
                              rcv_sem.at[p]).wait()

    def idx_compute(p, i):
        r = r_v[p, pl.ds(i * L, L)]
        c = c_v[p, pl.ds(i * L, L)]
        q1 = lax.shift_right_logical(r, 4)
        n = (q1.astype(jnp.float32) * 0.2).astype(jnp.int32)
        g = c * B + (r - n * B)
        g_v[p, i // NG, pl.ds((i % NG) * L, L)] = g
        n_v[p, i // NG, pl.ds((i % NG) * L, L)] = n

    def idx_pass(p):
        @plsc.parallel_loop(0, CHUNK // L, unroll=2)
        def _(i):
            idx_compute(p, i)

    def fire_gathers(p):
        for j in range(NG):
            pltpu.async_copy(g2_hbm.at[g_v.at[p, j]],
                             rows_v.at[p, pl.ds(j * GRP, GRP)],
                             g_sem.at[p])

    def drain_gathers(p):
        for j in range(NG):
            pltpu.make_async_copy(g2_hbm.at[g_v.at[p, j]],
                                  rows_v.at[p, pl.ds(j * GRP, GRP)],
                                  g_sem.at[p]).wait()

    def scale_pass(p):
        @plsc.parallel_loop(0, CHUNK // L, unroll=2)
        def _(i):
            vv = v_v[p, pl.ds(i * L, L)]
            for e in range(L):
                edge = i * L + e
                eb = jnp.broadcast_to(i * 0 + e, (L,)).astype(jnp.int32)
                sp = jnp.take_along_axis(vv, eb, axis=0)
                rows_v[p, edge, :] = rows_v[p, edge, :] * sp

    def fire_scatter(p):
        for j in range(NG):
            pltpu.async_copy(rows_v.at[p, pl.ds(j * GRP, GRP)],
                             acc_sh.at[n_v.at[p, j]],
                             s_sem.at[p], add=True)

    def drain_scatter(p):
        for j in range(NG):
            pltpu.make_async_copy(rows_v.at[p, pl.ds(j * GRP, GRP)],
                                  acc_sh.at[n_v.at[p, j]],
                                  s_sem.at[p]).wait()

    issue_rcv(0, 0)
    wait_rcv(0)
    idx_pass(0)
    fire_gathers(0)
    issue_rcv(1, 1)

    def loop_body(t, _):
        p = lax.rem(t, 2)
        q = 1 - p

        @pl.when(t >= 1)
        def _():
            drain_scatter(q)

        @pl.when(t < nch - 1)
        def _():
            wait_rcv(q)
            idx_pass(q)

        drain_gathers(p)

        @pl.when(t < nch - 1)
        def _():
            fire_gathers(q)

        scale_pass(p)
        fire_scatter(p)

        @pl.when(t < nch - 2)
        def _():
            issue_rcv(t + 2, p)
        return 0

    lax.fori_loop(0, nch, loop_body, 0)
    drain_scatter(lax.rem(nch - 1, 2))

    plsc.subcore_barrier()
    pltpu.sync_copy(acc_sh.at[pl.ds(sid * NPS, NPS)],
                    out_hbm.at[cid, pl.ds(sid * NPS, NPS)])

    @pl.when(sid == 0)
    def _():
        pltpu.sync_copy(acc_sh.at[pl.ds(NPS * NS, NTAIL)],
                        out_hbm.at[cid, pl.ds(NPS * NS, NTAIL)])


@functools.partial(
    pl.kernel,
    out_type=jax.ShapeDtypeStruct((NC, N, F_OUT), jnp.float32),
    mesh=plsc.VectorSubcoreMesh(core_axis_name="c", subcore_axis_name="s"),
    compiler_params=pltpu.CompilerParams(needs_layout_passes=False,
                                         use_tc_tiling_on_sc=False),
    scratch_types=[
        pltpu.VMEM((2, CHUNK), jnp.int32),
        pltpu.VMEM((2, CHUNK), jnp.int32),
        pltpu.VMEM((2, CHUNK), jnp.float32),
        pltpu.VMEM((2, NG, GRP), jnp.int32),
        pltpu.VMEM((2, NG, GRP), jnp.int32),
        pltpu.VMEM((2, CHUNK, F_OUT), jnp.float32),
        pltpu.VMEM_SHARED((N, F_OUT), jnp.float32),
        pltpu.SemaphoreType.DMA((2,)),
        pltpu.SemaphoreType.DMA((2,)),
        pltpu.SemaphoreType.DMA((2,)),
    ],
)
def _sparse_stage(g2_hbm, rows_hbm, cols_hbm, vals_hbm, zeros_hbm, out_hbm,
                  r_v, c_v, v_v, g_v, n_v, rows_v, acc_sh,
                  rcv_sem, g_sem, s_sem):
    _sc_body(g2_hbm, rows_hbm, cols_hbm, vals_hbm, zeros_hbm, out_hbm,
             r_v, c_v, v_v, g_v, n_v, rows_v, acc_sh, rcv_sem, g_sem, s_sem)


def kernel(x, conn_rows, conn_cols, conn_values, weights):
    m = _build_m(weights)
    g2 = _dense_stage(x, m).reshape(N * B, F_OUT)

    zeros = jnp.zeros((N, F_OUT), jnp.float32)
    partials = _sparse_stage(g2,
                             conn_rows.astype(jnp.int32),
                             conn_cols.astype(jnp.int32),
                             conn_values.astype(jnp.float32),
                             zeros)
    return _combine_stage(partials)
